# Initial kernel scaffold; baseline (speedup 1.0000x reference)
import jax
import jax.numpy as jnp
from jax.experimental import pallas as pl


def kernel(x, edge_index, W1a, b1a, W2a, b2a, W1b, b1b, W2b, b2b):
    raise NotImplementedError("write your pallas kernel here")


# speedup vs baseline: 5.5537x; 5.5537x over previous
import functools

import jax
import jax.numpy as jnp
from jax import lax
from jax.experimental import pallas as pl
from jax.experimental.pallas import tpu as pltpu
from jax.experimental.pallas import tpu_sc as plsc

NC, NS, L = 2, 16, 16
NW = NC * NS

D = 128
C = 80
ZROWS = 64


def _encode_body(x_ref, w_ref, b_ref, a_ref, bo_ref):
    xv = x_ref[...]
    a_ref[...] = (
        jnp.dot(xv, w_ref[0:D, :], preferred_element_type=jnp.float32)
        + b_ref[...]
    )
    bo_ref[...] = jnp.dot(xv, w_ref[D : 2 * D, :], preferred_element_type=jnp.float32)


def _encode(x, w1, b1):
    n = x.shape[0]
    bm = 400
    grid = (pl.cdiv(n, bm),)
    return pl.pallas_call(
        _encode_body,
        grid=grid,
        in_specs=[
            pl.BlockSpec((bm, D), lambda i: (i, 0)),
            pl.BlockSpec((2 * D, D), lambda i: (0, 0)),
            pl.BlockSpec((1, D), lambda i: (0, 0)),
        ],
        out_specs=[
            pl.BlockSpec((bm, D), lambda i: (i, 0)),
            pl.BlockSpec((bm, D), lambda i: (i, 0)),
        ],
        out_shape=[
            jax.ShapeDtypeStruct((n, D), jnp.float32),
            jax.ShapeDtypeStruct((n, D), jnp.float32),
        ],
    )(x, w1, b1.reshape(1, D))


def _decode_body(sp_ref, cnt_ref, w_ref, b_ref, o_ref, *, apply_relu):
    s = sp_ref[0] + sp_ref[1]
    c = cnt_ref[0] + cnt_ref[1]
    t = jnp.dot(s, w_ref[...], preferred_element_type=jnp.float32)
    r = 1.0 / jnp.maximum(c, 1.0)
    o = t * r + jnp.where(c > 0.0, 1.0, 0.0) * b_ref[...]
    if apply_relu:
        o = jnp.maximum(o, 0.0)
    o_ref[...] = o


def _decode(sp, cnt_col, w2, b2, apply_relu):
    n = sp.shape[1]
    bm = 400
    grid = (pl.cdiv(n, bm),)
    return pl.pallas_call(
        functools.partial(_decode_body, apply_relu=apply_relu),
        grid=grid,
        in_specs=[
            pl.BlockSpec((2, bm, D), lambda i: (0, i, 0)),
            pl.BlockSpec((2, bm, 1), lambda i: (0, i, 0)),
            pl.BlockSpec((D, D), lambda i: (0, 0)),
            pl.BlockSpec((1, D), lambda i: (0, 0)),
        ],
        out_specs=pl.BlockSpec((bm, D), lambda i: (i, 0)),
        out_shape=jax.ShapeDtypeStruct((n, D), jnp.float32),
    )(sp, cnt_col, w2, b2.reshape(1, D))


def _make_edge_pass(n, e):
    ept = e // NW
    chunks = ept // C
    m = pl.cdiv(n, NS * 8) * NS * 8
    rpt = m // NS
    crows = pl.cdiv(m, D)
    crows = pl.cdiv(crows, 8) * 8
    mesh = plsc.VectorSubcoreMesh(
        core_axis_name="c", subcore_axis_name="s", num_cores=NC, num_subcores=NS
    )

    @functools.partial(
        pl.kernel,
        mesh=mesh,
        out_type=(
            jax.ShapeDtypeStruct((NC, m, D), jnp.float32),
            jax.ShapeDtypeStruct((NC, crows, D), jnp.float32),
        ),
        scratch_types=[
            pltpu.VMEM((C,), jnp.int32),
            pltpu.VMEM((C,), jnp.int32),
            pltpu.VMEM((C, D), jnp.float32),
            pltpu.VMEM((C, D), jnp.float32),
            pltpu.VMEM((ZROWS, D), jnp.float32),
            pltpu.VMEM((crows, D), jnp.float32),
            pltpu.VMEM((crows,), jnp.int32),
            pltpu.VMEM_SHARED((m, D), jnp.float32),
            pltpu.VMEM_SHARED((crows, D), jnp.float32),
            pltpu.SemaphoreType.DMA,
            pltpu.SemaphoreType.DMA,
        ],
        compiler_params=pltpu.CompilerParams(needs_layout_passes=False),
    )
    def edge_pass(a_hbm, b_hbm, src_hbm, dst_hbm, out_hbm, cnt_hbm,
                  dsti, srci, ga, gb, zbuf, cloc, rowids, s_acc, c_acc,
                  sem_a, sem_b):
        cid = lax.axis_index("c")
        sid = lax.axis_index("s")
        wid = sid * NC + cid

        zero16 = jnp.zeros((L,), jnp.float32)
        iota = lax.iota(jnp.int32, L)

        def zrow(i, carry):
            for j in range(D // L):
                zbuf[i, pl.ds(j * L, L)] = zero16
            return carry

        lax.fori_loop(0, ZROWS, zrow, 0)

        def crow(i, carry):
            for j in range(D // L):
                cloc[i, pl.ds(j * L, L)] = zero16
            return carry

        lax.fori_loop(0, crows, crow, 0)

        for k in range(crows // L):
            rowids[pl.ds(k * L, L)] = iota + k * L

        row0 = sid * rpt
        for k in range(rpt // ZROWS):
            pltpu.sync_copy(zbuf, s_acc.at[pl.ds(row0 + k * ZROWS, ZROWS)])
        rem = rpt % ZROWS
        if rem:
            pltpu.sync_copy(
                zbuf.at[pl.ds(0, rem)],
                s_acc.at[pl.ds(row0 + (rpt // ZROWS) * ZROWS, rem)],
            )
        czslices = crows // 8
        @pl.when(sid < czslices)
        def _():
            pltpu.sync_copy(zbuf.at[pl.ds(0, 8)], c_acc.at[pl.ds(8 * sid, 8)])

        plsc.subcore_barrier()

        ebase = wid * ept

        def chunk(j, carry):
            base = pl.multiple_of(ebase + j * C, 8)
            pltpu.sync_copy(dst_hbm.at[pl.ds(base, C)], dsti)
            pltpu.sync_copy(src_hbm.at[pl.ds(base, C)], srci)
            cpa = pltpu.async_copy(a_hbm.at[dsti], ga, sem_a)
            cpb = pltpu.async_copy(b_hbm.at[srci], gb, sem_b)
            cpa.wait()
            cpb.wait()

            def row(i, inner):
                for j8 in range(D // L):
                    sl = pl.ds(j8 * L, L)
                    ga[i, sl] = jnp.maximum(ga[i, sl] + gb[i, sl], 0.0)
                return inner

            lax.fori_loop(0, C, row, 0)
            pltpu.sync_copy(ga, s_acc.at[dsti], add=True)

            ones = jnp.ones((L,), jnp.float32)
            for k in range(C // L):
                dv = dsti[pl.ds(k * L, L)]
                plsc.addupdate_scatter(
                    cloc,
                    [lax.shift_right_logical(dv, 7), lax.bitwise_and(dv, 127)],
                    ones,
                )
            return carry

        lax.fori_loop(0, chunks, chunk, 0)

        pltpu.sync_copy(cloc, c_acc.at[rowids], add=True)
        plsc.subcore_barrier()

        pltpu.sync_copy(
            s_acc.at[pl.ds(row0, rpt)], out_hbm.at[cid, pl.ds(row0, rpt)]
        )
        @pl.when(sid < czslices)
        def _():
            pltpu.sync_copy(
                c_acc.at[pl.ds(8 * sid, 8)], cnt_hbm.at[cid, pl.ds(8 * sid, 8)]
            )

    return edge_pass, m, crows


def kernel(x, edge_index, W1a, b1a, W2a, b2a, W1b, b1b, W2b, b2b):
    n = x.shape[0]
    e = edge_index.shape[1]
    src = edge_index[0]
    dst = edge_index[1]

    edge_pass, m, crows = _make_edge_pass(n, e)

    a1, bb1 = _encode(x, W1a, b1a)
    sp1, cnt1 = edge_pass(a1, bb1, src, dst)
    h = _decode(sp1, cnt1.reshape(NC, crows * D, 1), W2a, b2a, apply_relu=True)

    a2, bb2 = _encode(h, W1b, b1b)
    sp2, cnt2 = edge_pass(a2, bb2, src, dst)
    out = _decode(sp2, cnt2.reshape(NC, crows * D, 1), W2b, b2b, apply_relu=False)
    return out[:n]


# speedup vs baseline: 7.7580x; 1.3969x over previous
import functools

import jax
import jax.numpy as jnp
from jax import lax
from jax.experimental import pallas as pl
from jax.experimental.pallas import tpu as pltpu
from jax.experimental.pallas import tpu_sc as plsc

NC, NS, L = 2, 16, 16
NW = NC * NS

D = 128
C = 64
ZROWS = 8


def _encode_body(x_ref, w_ref, b_ref, a_ref, bo_ref):
    xv = x_ref[...]
    a_ref[...] = (
        jnp.dot(xv, w_ref[0:D, :], preferred_element_type=jnp.float32)
        + b_ref[...]
    )
    bo_ref[...] = jnp.dot(xv, w_ref[D : 2 * D, :], preferred_element_type=jnp.float32)


def _encode(x, w1, b1):
    n = x.shape[0]
    bm = 416
    grid = (pl.cdiv(n, bm),)
    return pl.pallas_call(
        _encode_body,
        grid=grid,
        in_specs=[
            pl.BlockSpec((bm, D), lambda i: (i, 0)),
            pl.BlockSpec((2 * D, D), lambda i: (0, 0)),
            pl.BlockSpec((1, D), lambda i: (0, 0)),
        ],
        out_specs=[
            pl.BlockSpec((bm, D), lambda i: (i, 0)),
            pl.BlockSpec((bm, D), lambda i: (i, 0)),
        ],
        out_shape=[
            jax.ShapeDtypeStruct((n, D), jnp.float32),
            jax.ShapeDtypeStruct((n, D), jnp.float32),
        ],
    )(x, w1, b1.reshape(1, D))


def _decode_body(sp_ref, cnt_ref, w_ref, b_ref, o_ref, *, apply_relu):
    s = sp_ref[0] + sp_ref[1]
    c = cnt_ref[0] + cnt_ref[1]
    t = jnp.dot(s, w_ref[...], preferred_element_type=jnp.float32)
    r = 1.0 / jnp.maximum(c, 1.0)
    o = t * r + jnp.where(c > 0.0, 1.0, 0.0) * b_ref[...]
    if apply_relu:
        o = jnp.maximum(o, 0.0)
    o_ref[...] = o


def _decode(sp, cnt_col, w2, b2, apply_relu):
    n = sp.shape[1]
    bm = 416
    grid = (pl.cdiv(n, bm),)
    return pl.pallas_call(
        functools.partial(_decode_body, apply_relu=apply_relu),
        grid=grid,
        in_specs=[
            pl.BlockSpec((2, bm, D), lambda i: (0, i, 0)),
            pl.BlockSpec((2, bm, 1), lambda i: (0, i, 0)),
            pl.BlockSpec((D, D), lambda i: (0, 0)),
            pl.BlockSpec((1, D), lambda i: (0, 0)),
        ],
        out_specs=pl.BlockSpec((bm, D), lambda i: (i, 0)),
        out_shape=jax.ShapeDtypeStruct((n, D), jnp.float32),
    )(sp, cnt_col, w2, b2.reshape(1, D))


def _make_edge_pass(n, chunks):
    m = pl.cdiv(n + 1, NS * 8) * NS * 8
    rpt = m // NS
    crows = pl.cdiv(pl.cdiv(m, D), 8) * 8
    mesh = plsc.VectorSubcoreMesh(
        core_axis_name="c", subcore_axis_name="s", num_cores=NC, num_subcores=NS
    )

    @functools.partial(
        pl.kernel,
        mesh=mesh,
        out_type=(
            jax.ShapeDtypeStruct((NC, m, D), jnp.float32),
            jax.ShapeDtypeStruct((NC, crows, D), jnp.float32),
        ),
        scratch_types=[
            [pltpu.VMEM((C,), jnp.int32)] * 4,
            [pltpu.VMEM((C,), jnp.int32)] * 4,
            [pltpu.VMEM((C, D), jnp.float32)] * 2,
            [pltpu.VMEM((C, D), jnp.float32)] * 2,
            pltpu.VMEM((ZROWS, D), jnp.float32),
            pltpu.VMEM((crows, D), jnp.float32),
            pltpu.VMEM((crows,), jnp.int32),
            pltpu.VMEM_SHARED((m, D), jnp.float32),
            pltpu.VMEM_SHARED((crows, D), jnp.float32),
            [pltpu.SemaphoreType.DMA] * 4,
            [pltpu.SemaphoreType.DMA] * 4,
            [pltpu.SemaphoreType.DMA] * 2,
            [pltpu.SemaphoreType.DMA] * 2,
        ],
        compiler_params=pltpu.CompilerParams(needs_layout_passes=False),
    )
    def edge_pass(a_hbm, b_hbm, src_hbm, dst_hbm, out_hbm, cnt_hbm,
                  dsti, srci, ga, gb, zbuf, cloc, rowids, s_acc, c_acc,
                  sem_di, sem_si, sem_a, sem_b):
        cid = lax.axis_index("c")
        sid = lax.axis_index("s")
        wid = sid * NC + cid

        zero16 = jnp.zeros((L,), jnp.float32)
        iota = lax.iota(jnp.int32, L)

        def zrow(i, carry):
            for j in range(D // L):
                zbuf[i, pl.ds(j * L, L)] = zero16
            return carry

        lax.fori_loop(0, ZROWS, zrow, 0)

        def crow(i, carry):
            for j in range(D // L):
                cloc[i, pl.ds(j * L, L)] = zero16
            return carry

        lax.fori_loop(0, crows, crow, 0)

        for k in range(crows // L):
            rowids[pl.ds(k * L, L)] = iota + k * L

        row0 = sid * rpt
        for k in range(rpt // ZROWS):
            pltpu.sync_copy(zbuf, s_acc.at[pl.ds(row0 + k * ZROWS, ZROWS)])
        czslices = crows // 8
        @pl.when(sid < czslices)
        def _():
            pltpu.sync_copy(zbuf.at[pl.ds(0, 8)], c_acc.at[pl.ds(8 * sid, 8)])

        plsc.subcore_barrier()

        crow0 = wid * chunks

        def fetch_idx(j, ib):
            @pl.when(j < chunks)
            def _():
                row = crow0 + j
                pltpu.async_copy(dst_hbm.at[row], dsti[ib], sem_di[ib])
                pltpu.async_copy(src_hbm.at[row], srci[ib], sem_si[ib])

        def fetch_gather(ib, b):
            pltpu.make_async_copy(dst_hbm.at[crow0], dsti[ib], sem_di[ib]).wait()
            pltpu.make_async_copy(src_hbm.at[crow0], srci[ib], sem_si[ib]).wait()
            pltpu.async_copy(a_hbm.at[dsti[ib]], ga[b], sem_a[b])
            pltpu.async_copy(b_hbm.at[srci[ib]], gb[b], sem_b[b])

        def wait_gather(ib, b):
            pltpu.make_async_copy(a_hbm.at[dsti[ib]], ga[b], sem_a[b]).wait()
            pltpu.make_async_copy(b_hbm.at[srci[ib]], gb[b], sem_b[b]).wait()

        ones = jnp.ones((L,), jnp.float32)

        def process(ib, b):
            def row(i, inner):
                for j8 in range(D // L):
                    sl = pl.ds(j8 * L, L)
                    ga[b][i, sl] = jnp.maximum(ga[b][i, sl] + gb[b][i, sl], 0.0)
                return inner

            lax.fori_loop(0, C, row, 0)

            for k in range(C // L):
                dv = dsti[ib][pl.ds(k * L, L)]
                plsc.addupdate_scatter(
                    cloc,
                    [lax.shift_right_logical(dv, 7), lax.bitwise_and(dv, 127)],
                    ones,
                )
            pltpu.sync_copy(ga[b], s_acc.at[dsti[ib]], add=True)

        assert chunks % 4 == 1 and chunks >= 5
        fetch_idx(0, 0)
        fetch_idx(1, 1)
        fetch_idx(2, 2)
        fetch_gather(0, 0)

        def pipe(kk, carry):
            j0 = kk * 4
            for u in range(4):
                j = j0 + u
                ib, b = u % 4, u % 2
                wait_gather(ib, b)
                fetch_idx(j + 3, (u + 3) % 4)
                fetch_gather((u + 1) % 4, (u + 1) % 2)
                process(ib, b)
            return carry

        lax.fori_loop(0, chunks // 4, pipe, 0)
        wait_gather(0, 0)
        process(0, 0)

        pltpu.sync_copy(cloc, c_acc.at[rowids], add=True)
        plsc.subcore_barrier()

        pltpu.sync_copy(
            s_acc.at[pl.ds(row0, rpt)], out_hbm.at[cid, pl.ds(row0, rpt)]
        )
        @pl.when(sid < czslices)
        def _():
            pltpu.sync_copy(
                c_acc.at[pl.ds(8 * sid, 8)], cnt_hbm.at[cid, pl.ds(8 * sid, 8)]
            )

    return edge_pass, m, crows


def kernel(x, edge_index, W1a, b1a, W2a, b2a, W1b, b1b, W2b, b2b):
    n = x.shape[0]
    e = edge_index.shape[1]

    chunks = pl.cdiv(e, NW * C)
    while chunks % 4 != 1:
        chunks += 1
    e_pad = NW * C * chunks

    edge_pass, m, crows = _make_edge_pass(n, chunks)

    src = jnp.concatenate(
        [edge_index[0], jnp.zeros((e_pad - e,), edge_index.dtype)]
    ).reshape(NW * chunks, C)
    dst = jnp.concatenate(
        [edge_index[1], jnp.full((e_pad - e,), n, edge_index.dtype)]
    ).reshape(NW * chunks, C)

    xp = jnp.zeros((m, D), x.dtype).at[:n].set(x)

    a1, bb1 = _encode(xp, W1a, b1a)
    sp1, cnt1 = edge_pass(a1, bb1, src, dst)
    h = _decode(sp1, cnt1.reshape(NC, crows * D, 1), W2a, b2a, apply_relu=True)

    a2, bb2 = _encode(h, W1b, b1b)
    sp2, cnt2 = edge_pass(a2, bb2, src, dst)
    out = _decode(sp2, cnt2.reshape(NC, crows * D, 1), W2b, b2b, apply_relu=False)
    return out[:n]


# speedup vs baseline: 7.9538x; 1.0252x over previous
import functools

import jax
import jax.numpy as jnp
from jax import lax
from jax.experimental import pallas as pl
from jax.experimental.pallas import tpu as pltpu
from jax.experimental.pallas import tpu_sc as plsc

NC, NS, L = 2, 16, 16
NW = NC * NS

D = 128
C = 48
ZROWS = 64
DUMMY_SPREAD = 96


def _encode_body(x_ref, w_ref, b_ref, a_ref, bo_ref):
    xv = x_ref[...]
    a_ref[...] = (
        jnp.dot(xv, w_ref[0:D, :], preferred_element_type=jnp.float32)
        + b_ref[...]
    )
    bo_ref[...] = jnp.dot(xv, w_ref[D : 2 * D, :], preferred_element_type=jnp.float32)


def _encode(x, w1, b1):
    n = x.shape[0]
    bm = 416
    grid = (pl.cdiv(n, bm),)
    return pl.pallas_call(
        _encode_body,
        grid=grid,
        in_specs=[
            pl.BlockSpec((bm, D), lambda i: (i, 0)),
            pl.BlockSpec((2 * D, D), lambda i: (0, 0)),
            pl.BlockSpec((1, D), lambda i: (0, 0)),
        ],
        out_specs=[
            pl.BlockSpec((bm, D), lambda i: (i, 0)),
            pl.BlockSpec((bm, D), lambda i: (i, 0)),
        ],
        out_shape=[
            jax.ShapeDtypeStruct((n, D), jnp.float32),
            jax.ShapeDtypeStruct((n, D), jnp.float32),
        ],
    )(x, w1, b1.reshape(1, D))


def _decode_body(sp_ref, cnt_ref, w_ref, b_ref, o_ref, *, apply_relu):
    s = sp_ref[0] + sp_ref[1]
    c = cnt_ref[0] + cnt_ref[1]
    t = jnp.dot(s, w_ref[...], preferred_element_type=jnp.float32)
    r = 1.0 / jnp.maximum(c, 1.0)
    o = t * r + jnp.where(c > 0.0, 1.0, 0.0) * b_ref[...]
    if apply_relu:
        o = jnp.maximum(o, 0.0)
    o_ref[...] = o


def _decode(sp, cnt_col, w2, b2, apply_relu):
    n = sp.shape[1]
    bm = 416
    grid = (pl.cdiv(n, bm),)
    return pl.pallas_call(
        functools.partial(_decode_body, apply_relu=apply_relu),
        grid=grid,
        in_specs=[
            pl.BlockSpec((2, bm, D), lambda i: (0, i, 0)),
            pl.BlockSpec((2, bm, 1), lambda i: (0, i, 0)),
            pl.BlockSpec((D, D), lambda i: (0, 0)),
            pl.BlockSpec((1, D), lambda i: (0, 0)),
        ],
        out_specs=pl.BlockSpec((bm, D), lambda i: (i, 0)),
        out_shape=jax.ShapeDtypeStruct((n, D), jnp.float32),
    )(sp, cnt_col, w2, b2.reshape(1, D))


def _make_edge_pass(n, chunks, do_counts):
    m = pl.cdiv(n + DUMMY_SPREAD, NS * 8) * NS * 8
    rpt = m // NS
    crows = pl.cdiv(pl.cdiv(m, D), 8) * 8
    mesh = plsc.VectorSubcoreMesh(
        core_axis_name="c", subcore_axis_name="s", num_cores=NC, num_subcores=NS
    )

    out_type = [jax.ShapeDtypeStruct((NC, m, D), jnp.float32)]
    scratch = [
        [pltpu.VMEM((C,), jnp.int32)] * 8,
        [pltpu.VMEM((C,), jnp.int32)] * 8,
        [pltpu.VMEM((C, D), jnp.float32)] * 2,
        [pltpu.VMEM((C, D), jnp.float32)] * 2,
        [pltpu.VMEM((C, D), jnp.float32)] * 2,
        pltpu.VMEM_SHARED((m, D), jnp.float32),
        [pltpu.SemaphoreType.DMA] * 8,
        [pltpu.SemaphoreType.DMA] * 8,
        [pltpu.SemaphoreType.DMA] * 2,
        [pltpu.SemaphoreType.DMA] * 2,
        [pltpu.SemaphoreType.DMA] * 2,
    ]
    if do_counts:
        out_type.append(jax.ShapeDtypeStruct((NC, crows, D), jnp.float32))
        scratch += [
            pltpu.VMEM((crows, D), jnp.float32),
            pltpu.VMEM((crows,), jnp.int32),
            pltpu.VMEM_SHARED((crows, D), jnp.float32),
        ]

    @functools.partial(
        pl.kernel,
        mesh=mesh,
        out_type=tuple(out_type),
        scratch_types=scratch,
        compiler_params=pltpu.CompilerParams(needs_layout_passes=False),
    )
    def edge_pass(a_hbm, b_hbm, src_hbm, dst_hbm, zz_hbm, out_hbm, *rest):
        if do_counts:
            (cnt_hbm, dsti, srci, ga, gb, gv, s_acc,
             sem_di, sem_si, sem_a, sem_b, sem_sc, cloc, rowids, c_acc) = rest
        else:
            (dsti, srci, ga, gb, gv, s_acc,
             sem_di, sem_si, sem_a, sem_b, sem_sc) = rest
        cid = lax.axis_index("c")
        sid = lax.axis_index("s")
        wid = sid * NC + cid

        zero16 = jnp.zeros((L,), jnp.float32)
        iota = lax.iota(jnp.int32, L)

        row0 = sid * rpt
        for k in range(rpt // ZROWS):
            pltpu.sync_copy(zz_hbm, s_acc.at[pl.ds(row0 + k * ZROWS, ZROWS)])
        zrem = rpt % ZROWS
        if zrem:
            pltpu.sync_copy(
                zz_hbm.at[pl.ds(0, zrem)],
                s_acc.at[pl.ds(row0 + (rpt // ZROWS) * ZROWS, zrem)],
            )
        if do_counts:
            def crow(i, carry):
                for j in range(D // L):
                    cloc[i, pl.ds(j * L, L)] = zero16
                return carry

            lax.fori_loop(0, crows, crow, 0)
            for k in range(crows // L):
                rowids[pl.ds(k * L, L)] = iota + k * L
            czslices = crows // 8
            @pl.when(sid < czslices)
            def _():
                pltpu.sync_copy(
                    zz_hbm.at[pl.ds(0, 8)], c_acc.at[pl.ds(8 * sid, 8)]
                )

        plsc.subcore_barrier()

        def fetch_idx(j, ib):
            @pl.when(j < chunks)
            def _():
                row = j * NW + wid
                pltpu.async_copy(dst_hbm.at[row], dsti[ib], sem_di[ib])
                pltpu.async_copy(src_hbm.at[row], srci[ib], sem_si[ib])

        def fetch_gather(ib, b):
            pltpu.make_async_copy(dst_hbm.at[0], dsti[ib], sem_di[ib]).wait()
            pltpu.make_async_copy(src_hbm.at[0], srci[ib], sem_si[ib]).wait()
            pltpu.async_copy(a_hbm.at[dsti[ib]], ga[b], sem_a[b])
            pltpu.async_copy(b_hbm.at[srci[ib]], gb[b], sem_b[b])

        def wait_gather(ib, b):
            pltpu.make_async_copy(a_hbm.at[dsti[ib]], ga[b], sem_a[b]).wait()
            pltpu.make_async_copy(b_hbm.at[srci[ib]], gb[b], sem_b[b]).wait()

        def wait_scatter(ib, b):
            pltpu.make_async_copy(gv[b], s_acc.at[dsti[ib]], sem_sc[b]).wait()

        ones = jnp.ones((L,), jnp.float32)

        def process(j, ib, b):
            def row(i, inner):
                for j8 in range(D // L):
                    sl = pl.ds(j8 * L, L)
                    gv[b][i, sl] = jnp.maximum(ga[b][i, sl] + gb[b][i, sl], 0.0)
                return inner

            lax.fori_loop(0, C, row, 0)

            if do_counts:
                for k in range(C // L):
                    dv = dsti[ib][pl.ds(k * L, L)]
                    plsc.addupdate_scatter(
                        cloc,
                        [lax.shift_right_logical(dv, 7),
                         lax.bitwise_and(dv, 127)],
                        ones,
                    )
            pib, pb = (ib - 1) % 8, (b - 1) % 2
            @pl.when(j > 0)
            def _():
                wait_scatter(pib, pb)
            pltpu.async_copy(gv[b], s_acc.at[dsti[ib]], sem_sc[b], add=True)

        assert chunks % 8 == 1 and chunks >= 9
        fetch_idx(0, 0)
        fetch_idx(1, 1)
        fetch_idx(2, 2)
        fetch_gather(0, 0)

        def pipe(kk, carry):
            j0 = kk * 8
            for u in range(8):
                j = j0 + u
                ib, b = u, u % 2
                wait_gather(ib, b)
                fetch_idx(j + 3, (u + 3) % 8)
                fetch_gather((u + 1) % 8, (u + 1) % 2)
                process(j, ib, b)
            return carry

        lax.fori_loop(0, chunks // 8, pipe, 0)
        jt = (chunks // 8) * 8
        wait_gather(0, 0)
        process(jt, 0, 0)
        wait_scatter(0, 0)

        if do_counts:
            pltpu.sync_copy(cloc, c_acc.at[rowids], add=True)
        plsc.subcore_barrier()

        pltpu.sync_copy(
            s_acc.at[pl.ds(row0, rpt)], out_hbm.at[cid, pl.ds(row0, rpt)]
        )
        if do_counts:
            @pl.when(sid < crows // 8)
            def _():
                pltpu.sync_copy(
                    c_acc.at[pl.ds(8 * sid, 8)],
                    cnt_hbm.at[cid, pl.ds(8 * sid, 8)],
                )

    return edge_pass, m, crows


def kernel(x, edge_index, W1a, b1a, W2a, b2a, W1b, b1b, W2b, b2b):
    n = x.shape[0]
    e = edge_index.shape[1]

    chunks = pl.cdiv(e, NW * C)
    while chunks % 8 != 1:
        chunks += 1
    e_pad = NW * C * chunks

    edge_a, m, crows = _make_edge_pass(n, chunks, do_counts=True)
    edge_b, _, _ = _make_edge_pass(n, chunks, do_counts=False)

    npad = e_pad - e
    src = jnp.concatenate(
        [edge_index[0], jnp.zeros((npad,), edge_index.dtype)]
    ).reshape(NW * chunks, C)
    dst = jnp.concatenate(
        [edge_index[1],
         (n + jnp.arange(npad, dtype=edge_index.dtype) % DUMMY_SPREAD)]
    ).reshape(NW * chunks, C)

    xp = jnp.zeros((m, D), x.dtype).at[:n].set(x)
    zz = jnp.zeros((ZROWS, D), jnp.float32)

    a1, bb1 = _encode(xp, W1a, b1a)
    sp1, cnt1 = edge_a(a1, bb1, src, dst, zz)
    cnt_col = cnt1.reshape(NC, crows * D, 1)
    h = _decode(sp1, cnt_col, W2a, b2a, apply_relu=True)

    a2, bb2 = _encode(h, W1b, b1b)
    sp2 = edge_b(a2, bb2, src, dst, zz)
    if isinstance(sp2, (tuple, list)):
        (sp2,) = sp2
    out = _decode(sp2, cnt_col, W2b, b2b, apply_relu=False)
    return out[:n]


# speedup vs baseline: 8.2059x; 1.0317x over previous
import functools

import jax
import jax.numpy as jnp
from jax import lax
from jax.experimental import pallas as pl
from jax.experimental.pallas import tpu as pltpu
from jax.experimental.pallas import tpu_sc as plsc

NC, NS, L = 2, 16, 16
NW = NC * NS

D = 128
C = 48
ZROWS = 64
DUMMY_SPREAD = 96


def _encode_body(x_ref, w_ref, b_ref, a_ref, bo_ref):
    xv = x_ref[...]
    a_ref[...] = (
        jnp.dot(xv, w_ref[0:D, :], preferred_element_type=jnp.float32)
        + b_ref[...]
    )
    bo_ref[...] = jnp.dot(xv, w_ref[D : 2 * D, :], preferred_element_type=jnp.float32)


def _encode(x, w1, b1, rows):
    bm = 416
    grid = (pl.cdiv(rows, bm),)
    return pl.pallas_call(
        _encode_body,
        grid=grid,
        in_specs=[
            pl.BlockSpec((bm, D), lambda i: (i, 0)),
            pl.BlockSpec((2 * D, D), lambda i: (0, 0)),
            pl.BlockSpec((1, D), lambda i: (0, 0)),
        ],
        out_specs=[
            pl.BlockSpec((bm, D), lambda i: (i, 0)),
            pl.BlockSpec((bm, D), lambda i: (i, 0)),
        ],
        out_shape=[
            jax.ShapeDtypeStruct((rows, D), jnp.float32),
            jax.ShapeDtypeStruct((rows, D), jnp.float32),
        ],
    )(x, w1, b1.reshape(1, D))


def _decode_body(sp_ref, cnt_ref, w_ref, b_ref, o_ref, *, apply_relu):
    s = sp_ref[0] + sp_ref[1]
    c = cnt_ref[0] + cnt_ref[1]
    t = jnp.dot(s, w_ref[...], preferred_element_type=jnp.float32)
    r = 1.0 / jnp.maximum(c, 1.0)
    o = t * r + jnp.where(c > 0.0, 1.0, 0.0) * b_ref[...]
    if apply_relu:
        o = jnp.maximum(o, 0.0)
    o_ref[...] = o


def _decode(sp, cnt_col, w2, b2, rows):
    bm = 416
    grid = (pl.cdiv(rows, bm),)
    return pl.pallas_call(
        functools.partial(_decode_body, apply_relu=False),
        grid=grid,
        in_specs=[
            pl.BlockSpec((2, bm, D), lambda i: (0, i, 0)),
            pl.BlockSpec((2, bm, 1), lambda i: (0, i, 0)),
            pl.BlockSpec((D, D), lambda i: (0, 0)),
            pl.BlockSpec((1, D), lambda i: (0, 0)),
        ],
        out_specs=pl.BlockSpec((bm, D), lambda i: (i, 0)),
        out_shape=jax.ShapeDtypeStruct((rows, D), jnp.float32),
    )(sp, cnt_col, w2, b2.reshape(1, D))


def _mid_body(sp_ref, cnt_ref, w2_ref, b2_ref, w1_ref, b1_ref, a_ref, bo_ref):
    s = sp_ref[0] + sp_ref[1]
    c = cnt_ref[0] + cnt_ref[1]
    t = jnp.dot(s, w2_ref[...], preferred_element_type=jnp.float32)
    r = 1.0 / jnp.maximum(c, 1.0)
    h = jnp.maximum(t * r + jnp.where(c > 0.0, 1.0, 0.0) * b2_ref[...], 0.0)
    a_ref[...] = (
        jnp.dot(h, w1_ref[0:D, :], preferred_element_type=jnp.float32)
        + b1_ref[...]
    )
    bo_ref[...] = jnp.dot(
        h, w1_ref[D : 2 * D, :], preferred_element_type=jnp.float32
    )


def _mid(sp, cnt_col, w2, b2, w1, b1, rows):
    bm = 416
    grid = (pl.cdiv(rows, bm),)
    return pl.pallas_call(
        _mid_body,
        grid=grid,
        in_specs=[
            pl.BlockSpec((2, bm, D), lambda i: (0, i, 0)),
            pl.BlockSpec((2, bm, 1), lambda i: (0, i, 0)),
            pl.BlockSpec((D, D), lambda i: (0, 0)),
            pl.BlockSpec((1, D), lambda i: (0, 0)),
            pl.BlockSpec((2 * D, D), lambda i: (0, 0)),
            pl.BlockSpec((1, D), lambda i: (0, 0)),
        ],
        out_specs=[
            pl.BlockSpec((bm, D), lambda i: (i, 0)),
            pl.BlockSpec((bm, D), lambda i: (i, 0)),
        ],
        out_shape=[
            jax.ShapeDtypeStruct((rows, D), jnp.float32),
            jax.ShapeDtypeStruct((rows, D), jnp.float32),
        ],
    )(sp, cnt_col, w2, b2.reshape(1, D), w1, b1.reshape(1, D))


def _make_edge_pass(n, chunks, do_counts):
    m = pl.cdiv(n + DUMMY_SPREAD, NS * 8) * NS * 8
    rpt = m // NS
    crows = pl.cdiv(pl.cdiv(m, D), 8) * 8
    mesh = plsc.VectorSubcoreMesh(
        core_axis_name="c", subcore_axis_name="s", num_cores=NC, num_subcores=NS
    )

    out_type = [jax.ShapeDtypeStruct((NC, m, D), jnp.float32)]
    scratch = [
        [pltpu.VMEM((C,), jnp.int32)] * 8,
        [pltpu.VMEM((C,), jnp.int32)] * 8,
        [pltpu.VMEM((C, D), jnp.float32)] * 2,
        [pltpu.VMEM((C, D), jnp.float32)] * 2,
        [pltpu.VMEM((C, D), jnp.float32)] * 2,
        pltpu.VMEM_SHARED((m, D), jnp.float32),
        [pltpu.SemaphoreType.DMA] * 8,
        [pltpu.SemaphoreType.DMA] * 8,
        [pltpu.SemaphoreType.DMA] * 2,
        [pltpu.SemaphoreType.DMA] * 2,
        [pltpu.SemaphoreType.DMA] * 2,
    ]
    if do_counts:
        out_type.append(jax.ShapeDtypeStruct((NC, crows, D), jnp.float32))
        scratch += [
            pltpu.VMEM((crows, D), jnp.float32),
            pltpu.VMEM((crows,), jnp.int32),
            pltpu.VMEM_SHARED((crows, D), jnp.float32),
        ]

    @functools.partial(
        pl.kernel,
        mesh=mesh,
        out_type=tuple(out_type),
        scratch_types=scratch,
        compiler_params=pltpu.CompilerParams(needs_layout_passes=False),
    )
    def edge_pass(a_hbm, b_hbm, src_hbm, dst_hbm, zz_hbm, out_hbm, *rest):
        if do_counts:
            (cnt_hbm, dsti, srci, ga, gb, gv, s_acc,
             sem_di, sem_si, sem_a, sem_b, sem_sc, cloc, rowids, c_acc) = rest
        else:
            (dsti, srci, ga, gb, gv, s_acc,
             sem_di, sem_si, sem_a, sem_b, sem_sc) = rest
        cid = lax.axis_index("c")
        sid = lax.axis_index("s")
        wid = sid * NC + cid

        zero16 = jnp.zeros((L,), jnp.float32)
        iota = lax.iota(jnp.int32, L)

        row0 = sid * rpt
        for k in range(rpt // ZROWS):
            pltpu.sync_copy(zz_hbm, s_acc.at[pl.ds(row0 + k * ZROWS, ZROWS)])
        zrem = rpt % ZROWS
        if zrem:
            pltpu.sync_copy(
                zz_hbm.at[pl.ds(0, zrem)],
                s_acc.at[pl.ds(row0 + (rpt // ZROWS) * ZROWS, zrem)],
            )
        if do_counts:
            def crow(i, carry):
                for j in range(D // L):
                    cloc[i, pl.ds(j * L, L)] = zero16
                return carry

            lax.fori_loop(0, crows, crow, 0)
            for k in range(crows // L):
                rowids[pl.ds(k * L, L)] = iota + k * L
            czslices = crows // 8
            @pl.when(sid < czslices)
            def _():
                pltpu.sync_copy(
                    zz_hbm.at[pl.ds(0, 8)], c_acc.at[pl.ds(8 * sid, 8)]
                )

        plsc.subcore_barrier()

        def fetch_idx(j, ib):
            @pl.when(j < chunks)
            def _():
                row = j * NW + wid
                pltpu.async_copy(dst_hbm.at[row], dsti[ib], sem_di[ib])
                pltpu.async_copy(src_hbm.at[row], srci[ib], sem_si[ib])

        def fetch_gather(ib, b):
            pltpu.make_async_copy(dst_hbm.at[0], dsti[ib], sem_di[ib]).wait()
            pltpu.make_async_copy(src_hbm.at[0], srci[ib], sem_si[ib]).wait()
            pltpu.async_copy(a_hbm.at[dsti[ib]], ga[b], sem_a[b])
            pltpu.async_copy(b_hbm.at[srci[ib]], gb[b], sem_b[b])

        def wait_gather(ib, b):
            pltpu.make_async_copy(a_hbm.at[dsti[ib]], ga[b], sem_a[b]).wait()
            pltpu.make_async_copy(b_hbm.at[srci[ib]], gb[b], sem_b[b]).wait()

        def wait_scatter(ib, b):
            pltpu.make_async_copy(gv[b], s_acc.at[dsti[ib]], sem_sc[b]).wait()

        ones = jnp.ones((L,), jnp.float32)

        def process(j, ib, b):
            def row(i, inner):
                for j8 in range(D // L):
                    sl = pl.ds(j8 * L, L)
                    gv[b][i, sl] = jnp.maximum(ga[b][i, sl] + gb[b][i, sl], 0.0)
                return inner

            lax.fori_loop(0, C, row, 0)

            if do_counts:
                for k in range(C // L):
                    dv = dsti[ib][pl.ds(k * L, L)]
                    plsc.addupdate_scatter(
                        cloc,
                        [lax.shift_right_logical(dv, 7),
                         lax.bitwise_and(dv, 127)],
                        ones,
                    )
            pib, pb = (ib - 1) % 8, (b - 1) % 2
            @pl.when(j > 0)
            def _():
                wait_scatter(pib, pb)
            pltpu.async_copy(gv[b], s_acc.at[dsti[ib]], sem_sc[b], add=True)

        assert chunks % 8 == 1 and chunks >= 9
        fetch_idx(0, 0)
        fetch_idx(1, 1)
        fetch_idx(2, 2)
        fetch_gather(0, 0)

        def pipe(kk, carry):
            j0 = kk * 8
            for u in range(8):
                j = j0 + u
                ib, b = u, u % 2
                wait_gather(ib, b)
                fetch_idx(j + 3, (u + 3) % 8)
                fetch_gather((u + 1) % 8, (u + 1) % 2)
                process(j, ib, b)
            return carry

        lax.fori_loop(0, chunks // 8, pipe, 0)
        jt = (chunks // 8) * 8
        wait_gather(0, 0)
        process(jt, 0, 0)
        wait_scatter(0, 0)

        if do_counts:
            pltpu.sync_copy(cloc, c_acc.at[rowids], add=True)
        plsc.subcore_barrier()

        pltpu.sync_copy(
            s_acc.at[pl.ds(row0, rpt)], out_hbm.at[cid, pl.ds(row0, rpt)]
        )
        if do_counts:
            @pl.when(sid < crows // 8)
            def _():
                pltpu.sync_copy(
                    c_acc.at[pl.ds(8 * sid, 8)],
                    cnt_hbm.at[cid, pl.ds(8 * sid, 8)],
                )

    return edge_pass, m, crows


def kernel(x, edge_index, W1a, b1a, W2a, b2a, W1b, b1b, W2b, b2b):
    n = x.shape[0]
    e = edge_index.shape[1]

    chunks = pl.cdiv(e, NW * C)
    while chunks % 8 != 1:
        chunks += 1
    e_pad = NW * C * chunks

    edge_a, m, crows = _make_edge_pass(n, chunks, do_counts=True)
    edge_b, _, _ = _make_edge_pass(n, chunks, do_counts=False)

    npad = e_pad - e
    src = jnp.concatenate(
        [edge_index[0], jnp.zeros((npad,), edge_index.dtype)]
    ).reshape(NW * chunks, C)
    dst = jnp.concatenate(
        [edge_index[1],
         (n + jnp.arange(npad, dtype=edge_index.dtype) % DUMMY_SPREAD)]
    ).reshape(NW * chunks, C)

    zz = jnp.zeros((ZROWS, D), jnp.float32)

    a1, bb1 = _encode(x, W1a, b1a, m)
    sp1, cnt1 = edge_a(a1, bb1, src, dst, zz)
    cnt_col = cnt1.reshape(NC, crows * D, 1)
    a2, bb2 = _mid(sp1, cnt_col, W2a, b2a, W1b, b1b, m)

    sp2 = edge_b(a2, bb2, src, dst, zz)
    if isinstance(sp2, (tuple, list)):
        (sp2,) = sp2
    return _decode(sp2, cnt_col, W2b, b2b, n)


# speedup vs baseline: 10.8200x; 1.3186x over previous
import functools

import jax
import jax.numpy as jnp
from jax import lax
from jax.experimental import pallas as pl
from jax.experimental.pallas import tpu as pltpu
from jax.experimental.pallas import tpu_sc as plsc

NC, NS, L = 2, 16, 16
NW = NC * NS

D = 128
C = 32
ZROWS = 64
DUMMY_SPREAD = 96


def _encode_body(x_ref, w_ref, b_ref, a_ref, bo_ref):
    xv = x_ref[...]
    a_ref[...] = (
        jnp.dot(xv, w_ref[0:D, :], preferred_element_type=jnp.float32)
        + b_ref[...]
    )
    bo_ref[...] = jnp.dot(xv, w_ref[D : 2 * D, :], preferred_element_type=jnp.float32)


def _encode(x, w1, b1, rows):
    bm = 416
    grid = (pl.cdiv(rows, bm),)
    return pl.pallas_call(
        _encode_body,
        grid=grid,
        in_specs=[
            pl.BlockSpec((bm, D), lambda i: (i, 0)),
            pl.BlockSpec((2 * D, D), lambda i: (0, 0)),
            pl.BlockSpec((1, D), lambda i: (0, 0)),
        ],
        out_specs=[
            pl.BlockSpec((bm, D), lambda i: (i, 0)),
            pl.BlockSpec((bm, D), lambda i: (i, 0)),
        ],
        out_shape=[
            jax.ShapeDtypeStruct((rows, D), jnp.float32),
            jax.ShapeDtypeStruct((rows, D), jnp.float32),
        ],
    )(x, w1, b1.reshape(1, D))


def _decode_body(sp_ref, cnt_ref, w_ref, b_ref, o_ref, *, apply_relu):
    s = sp_ref[0] + sp_ref[1]
    c = cnt_ref[0] + cnt_ref[1]
    t = jnp.dot(s, w_ref[...], preferred_element_type=jnp.float32)
    r = 1.0 / jnp.maximum(c, 1.0)
    o = t * r + jnp.where(c > 0.0, 1.0, 0.0) * b_ref[...]
    if apply_relu:
        o = jnp.maximum(o, 0.0)
    o_ref[...] = o


def _decode(sp, cnt_col, w2, b2, rows):
    bm = 416
    grid = (pl.cdiv(rows, bm),)
    return pl.pallas_call(
        functools.partial(_decode_body, apply_relu=False),
        grid=grid,
        in_specs=[
            pl.BlockSpec((2, bm, D), lambda i: (0, i, 0)),
            pl.BlockSpec((2, bm, 1), lambda i: (0, i, 0)),
            pl.BlockSpec((D, D), lambda i: (0, 0)),
            pl.BlockSpec((1, D), lambda i: (0, 0)),
        ],
        out_specs=pl.BlockSpec((bm, D), lambda i: (i, 0)),
        out_shape=jax.ShapeDtypeStruct((rows, D), jnp.float32),
    )(sp, cnt_col, w2, b2.reshape(1, D))


def _mid_body(sp_ref, cnt_ref, w2_ref, b2_ref, w1_ref, b1_ref, a_ref, bo_ref):
    s = sp_ref[0] + sp_ref[1]
    c = cnt_ref[0] + cnt_ref[1]
    t = jnp.dot(s, w2_ref[...], preferred_element_type=jnp.float32)
    r = 1.0 / jnp.maximum(c, 1.0)
    h = jnp.maximum(t * r + jnp.where(c > 0.0, 1.0, 0.0) * b2_ref[...], 0.0)
    a_ref[...] = (
        jnp.dot(h, w1_ref[0:D, :], preferred_element_type=jnp.float32)
        + b1_ref[...]
    )
    bo_ref[...] = jnp.dot(
        h, w1_ref[D : 2 * D, :], preferred_element_type=jnp.float32
    )


def _mid(sp, cnt_col, w2, b2, w1, b1, rows):
    bm = 416
    grid = (pl.cdiv(rows, bm),)
    return pl.pallas_call(
        _mid_body,
        grid=grid,
        in_specs=[
            pl.BlockSpec((2, bm, D), lambda i: (0, i, 0)),
            pl.BlockSpec((2, bm, 1), lambda i: (0, i, 0)),
            pl.BlockSpec((D, D), lambda i: (0, 0)),
            pl.BlockSpec((1, D), lambda i: (0, 0)),
            pl.BlockSpec((2 * D, D), lambda i: (0, 0)),
            pl.BlockSpec((1, D), lambda i: (0, 0)),
        ],
        out_specs=[
            pl.BlockSpec((bm, D), lambda i: (i, 0)),
            pl.BlockSpec((bm, D), lambda i: (i, 0)),
        ],
        out_shape=[
            jax.ShapeDtypeStruct((rows, D), jnp.float32),
            jax.ShapeDtypeStruct((rows, D), jnp.float32),
        ],
    )(sp, cnt_col, w2, b2.reshape(1, D), w1, b1.reshape(1, D))


def _make_edge_pass(n, chunks, do_counts):
    m = pl.cdiv(n + DUMMY_SPREAD, NS * 8) * NS * 8
    rpt = m // NS
    crows = pl.cdiv(pl.cdiv(m, D), 8) * 8
    mesh = plsc.VectorSubcoreMesh(
        core_axis_name="c", subcore_axis_name="s", num_cores=NC, num_subcores=NS
    )

    out_type = [jax.ShapeDtypeStruct((NC, m, D), jnp.float32)]
    scratch = [
        [pltpu.VMEM((C,), jnp.int32)] * 8,
        [pltpu.VMEM((C,), jnp.int32)] * 8,
        [pltpu.VMEM((C, D), jnp.float32)] * 3,
        [pltpu.VMEM((C, D), jnp.float32)] * 3,
        [pltpu.VMEM((C, D), jnp.float32)] * 2,
        pltpu.VMEM_SHARED((m, D), jnp.float32),
        [pltpu.SemaphoreType.DMA] * 8,
        [pltpu.SemaphoreType.DMA] * 8,
        [pltpu.SemaphoreType.DMA] * 3,
        [pltpu.SemaphoreType.DMA] * 3,
        [pltpu.SemaphoreType.DMA] * 2,
    ]
    if do_counts:
        out_type.append(jax.ShapeDtypeStruct((NC, crows, D), jnp.float32))
        scratch += [
            pltpu.VMEM((crows, D), jnp.float32),
            pltpu.VMEM((crows,), jnp.int32),
            pltpu.VMEM_SHARED((crows, D), jnp.float32),
        ]

    @functools.partial(
        pl.kernel,
        mesh=mesh,
        out_type=tuple(out_type),
        scratch_types=scratch,
        compiler_params=pltpu.CompilerParams(needs_layout_passes=False),
    )
    def edge_pass(a_hbm, b_hbm, src_hbm, dst_hbm, zz_hbm, out_hbm, *rest):
        if do_counts:
            (cnt_hbm, dsti, srci, ga, gb, gv, s_acc,
             sem_di, sem_si, sem_a, sem_b, sem_sc, cloc, rowids, c_acc) = rest
        else:
            (dsti, srci, ga, gb, gv, s_acc,
             sem_di, sem_si, sem_a, sem_b, sem_sc) = rest
        cid = lax.axis_index("c")
        sid = lax.axis_index("s")
        wid = sid * NC + cid

        zero16 = jnp.zeros((L,), jnp.float32)
        iota = lax.iota(jnp.int32, L)

        row0 = sid * rpt
        for k in range(rpt // ZROWS):
            pltpu.sync_copy(zz_hbm, s_acc.at[pl.ds(row0 + k * ZROWS, ZROWS)])
        zrem = rpt % ZROWS
        if zrem:
            pltpu.sync_copy(
                zz_hbm.at[pl.ds(0, zrem)],
                s_acc.at[pl.ds(row0 + (rpt // ZROWS) * ZROWS, zrem)],
            )
        if do_counts:
            def crow(i, carry):
                for j in range(D // L):
                    cloc[i, pl.ds(j * L, L)] = zero16
                return carry

            lax.fori_loop(0, crows, crow, 0)
            for k in range(crows // L):
                rowids[pl.ds(k * L, L)] = iota + k * L
            czslices = crows // 8
            @pl.when(sid < czslices)
            def _():
                pltpu.sync_copy(
                    zz_hbm.at[pl.ds(0, 8)], c_acc.at[pl.ds(8 * sid, 8)]
                )

        plsc.subcore_barrier()

        def fetch_idx(j, ib):
            @pl.when(j < chunks)
            def _():
                row = j * NW + wid
                pltpu.async_copy(dst_hbm.at[row], dsti[ib], sem_di[ib])
                pltpu.async_copy(src_hbm.at[row], srci[ib], sem_si[ib])

        def fetch_gather(j, ib, g):
            @pl.when(j < chunks)
            def _():
                pltpu.make_async_copy(dst_hbm.at[0], dsti[ib], sem_di[ib]).wait()
                pltpu.make_async_copy(src_hbm.at[0], srci[ib], sem_si[ib]).wait()
                pltpu.async_copy(a_hbm.at[dsti[ib]], ga[g], sem_a[g])
                pltpu.async_copy(b_hbm.at[srci[ib]], gb[g], sem_b[g])

        def wait_gather(ib, g):
            pltpu.make_async_copy(a_hbm.at[dsti[ib]], ga[g], sem_a[g]).wait()
            pltpu.make_async_copy(b_hbm.at[srci[ib]], gb[g], sem_b[g]).wait()

        def wait_scatter(ib, b):
            pltpu.make_async_copy(gv[b], s_acc.at[dsti[ib]], sem_sc[b]).wait()

        ones = jnp.ones((L,), jnp.float32)

        def process(j, ib, g, b):
            def row(i, inner):
                for j8 in range(D // L):
                    sl = pl.ds(j8 * L, L)
                    gv[b][i, sl] = jnp.maximum(ga[g][i, sl] + gb[g][i, sl], 0.0)
                return inner

            lax.fori_loop(0, C, row, 0)

            if do_counts:
                for k in range(C // L):
                    dv = dsti[ib][pl.ds(k * L, L)]
                    plsc.addupdate_scatter(
                        cloc,
                        [lax.shift_right_logical(dv, 7),
                         lax.bitwise_and(dv, 127)],
                        ones,
                    )
            pib, pb = (ib - 1) % 8, (b - 1) % 2
            @pl.when(j > 0)
            def _():
                wait_scatter(pib, pb)
            pltpu.async_copy(gv[b], s_acc.at[dsti[ib]], sem_sc[b], add=True)

        assert chunks % 24 == 1 and chunks >= 25
        for p in range(4):
            fetch_idx(p, p)
        fetch_gather(0, 0, 0)
        fetch_gather(1, 1, 1)

        def pipe(kk, carry):
            j0 = kk * 24
            for u in range(24):
                j = j0 + u
                ib, g, b = u % 8, u % 3, u % 2
                wait_gather(ib, g)
                fetch_idx(j + 4, (u + 4) % 8)
                fetch_gather(j + 2, (u + 2) % 8, (u + 2) % 3)
                process(j, ib, g, b)
            return carry

        lax.fori_loop(0, chunks // 24, pipe, 0)
        jt = (chunks // 24) * 24
        wait_gather(0, 0)
        process(jt, 0, 0, 0)
        wait_scatter(0, 0)

        if do_counts:
            pltpu.sync_copy(cloc, c_acc.at[rowids], add=True)
        plsc.subcore_barrier()

        pltpu.sync_copy(
            s_acc.at[pl.ds(row0, rpt)], out_hbm.at[cid, pl.ds(row0, rpt)]
        )
        if do_counts:
            @pl.when(sid < crows // 8)
            def _():
                pltpu.sync_copy(
                    c_acc.at[pl.ds(8 * sid, 8)],
                    cnt_hbm.at[cid, pl.ds(8 * sid, 8)],
                )

    return edge_pass, m, crows


def kernel(x, edge_index, W1a, b1a, W2a, b2a, W1b, b1b, W2b, b2b):
    n = x.shape[0]
    e = edge_index.shape[1]

    chunks = pl.cdiv(e, NW * C)
    while chunks % 24 != 1:
        chunks += 1
    e_pad = NW * C * chunks

    edge_a, m, crows = _make_edge_pass(n, chunks, do_counts=True)
    edge_b, _, _ = _make_edge_pass(n, chunks, do_counts=False)

    npad = e_pad - e
    src = jnp.concatenate(
        [edge_index[0], jnp.zeros((npad,), edge_index.dtype)]
    ).reshape(NW * chunks, C)
    dst = jnp.concatenate(
        [edge_index[1],
         (n + jnp.arange(npad, dtype=edge_index.dtype) % DUMMY_SPREAD)]
    ).reshape(NW * chunks, C)

    zz = jnp.zeros((ZROWS, D), jnp.float32)

    a1, bb1 = _encode(x, W1a, b1a, m)
    sp1, cnt1 = edge_a(a1, bb1, src, dst, zz)
    cnt_col = cnt1.reshape(NC, crows * D, 1)
    a2, bb2 = _mid(sp1, cnt_col, W2a, b2a, W1b, b1b, m)

    sp2 = edge_b(a2, bb2, src, dst, zz)
    if isinstance(sp2, (tuple, list)):
        (sp2,) = sp2
    return _decode(sp2, cnt_col, W2b, b2b, n)


# speedup vs baseline: 11.1810x; 1.0334x over previous
import functools

import jax
import jax.numpy as jnp
from jax import lax
from jax.experimental import pallas as pl
from jax.experimental.pallas import tpu as pltpu
from jax.experimental.pallas import tpu_sc as plsc

NC, NS, L = 2, 16, 16
NW = NC * NS

D = 128
C = 32
ZROWS = 64
DUMMY_SPREAD = 96


def _encode_body(x_ref, w_ref, b_ref, a_ref, bo_ref):
    xv = x_ref[...]
    a_ref[...] = (
        jnp.dot(xv, w_ref[0:D, :], preferred_element_type=jnp.float32)
        + b_ref[...]
    )
    bo_ref[...] = jnp.dot(xv, w_ref[D : 2 * D, :], preferred_element_type=jnp.float32)


def _encode(x, w1, b1, rows):
    bm = 416
    grid = (pl.cdiv(rows, bm),)
    return pl.pallas_call(
        _encode_body,
        grid=grid,
        in_specs=[
            pl.BlockSpec((bm, D), lambda i: (i, 0)),
            pl.BlockSpec((2 * D, D), lambda i: (0, 0)),
            pl.BlockSpec((1, D), lambda i: (0, 0)),
        ],
        out_specs=[
            pl.BlockSpec((bm, D), lambda i: (i, 0)),
            pl.BlockSpec((bm, D), lambda i: (i, 0)),
        ],
        out_shape=[
            jax.ShapeDtypeStruct((rows, D), jnp.float32),
            jax.ShapeDtypeStruct((rows, D), jnp.float32),
        ],
    )(x, w1, b1.reshape(1, D))


def _decode_body(sp_ref, cnt_ref, w_ref, b_ref, o_ref, *, apply_relu):
    s = sp_ref[0] + sp_ref[1]
    c = cnt_ref[0] + cnt_ref[1]
    t = jnp.dot(s, w_ref[...], preferred_element_type=jnp.float32)
    r = 1.0 / jnp.maximum(c, 1.0)
    o = t * r + jnp.where(c > 0.0, 1.0, 0.0) * b_ref[...]
    if apply_relu:
        o = jnp.maximum(o, 0.0)
    o_ref[...] = o


def _decode(sp, cnt_col, w2, b2, rows):
    bm = 416
    grid = (pl.cdiv(rows, bm),)
    return pl.pallas_call(
        functools.partial(_decode_body, apply_relu=False),
        grid=grid,
        in_specs=[
            pl.BlockSpec((2, bm, D), lambda i: (0, i, 0)),
            pl.BlockSpec((2, bm, 1), lambda i: (0, i, 0)),
            pl.BlockSpec((D, D), lambda i: (0, 0)),
            pl.BlockSpec((1, D), lambda i: (0, 0)),
        ],
        out_specs=pl.BlockSpec((bm, D), lambda i: (i, 0)),
        out_shape=jax.ShapeDtypeStruct((rows, D), jnp.float32),
    )(sp, cnt_col, w2, b2.reshape(1, D))


def _mid_body(sp_ref, cnt_ref, w2_ref, b2_ref, w1_ref, b1_ref, a_ref, bo_ref):
    s = sp_ref[0] + sp_ref[1]
    c = cnt_ref[0] + cnt_ref[1]
    t = jnp.dot(s, w2_ref[...], preferred_element_type=jnp.float32)
    r = 1.0 / jnp.maximum(c, 1.0)
    h = jnp.maximum(t * r + jnp.where(c > 0.0, 1.0, 0.0) * b2_ref[...], 0.0)
    a_ref[...] = (
        jnp.dot(h, w1_ref[0:D, :], preferred_element_type=jnp.float32)
        + b1_ref[...]
    )
    bo_ref[...] = jnp.dot(
        h, w1_ref[D : 2 * D, :], preferred_element_type=jnp.float32
    )


def _mid(sp, cnt_col, w2, b2, w1, b1, rows):
    bm = 416
    grid = (pl.cdiv(rows, bm),)
    return pl.pallas_call(
        _mid_body,
        grid=grid,
        in_specs=[
            pl.BlockSpec((2, bm, D), lambda i: (0, i, 0)),
            pl.BlockSpec((2, bm, 1), lambda i: (0, i, 0)),
            pl.BlockSpec((D, D), lambda i: (0, 0)),
            pl.BlockSpec((1, D), lambda i: (0, 0)),
            pl.BlockSpec((2 * D, D), lambda i: (0, 0)),
            pl.BlockSpec((1, D), lambda i: (0, 0)),
        ],
        out_specs=[
            pl.BlockSpec((bm, D), lambda i: (i, 0)),
            pl.BlockSpec((bm, D), lambda i: (i, 0)),
        ],
        out_shape=[
            jax.ShapeDtypeStruct((rows, D), jnp.float32),
            jax.ShapeDtypeStruct((rows, D), jnp.float32),
        ],
    )(sp, cnt_col, w2, b2.reshape(1, D), w1, b1.reshape(1, D))


def _make_edge_pass(n, chunks, do_counts, gring, gahead, iahead, unroll):
    m = pl.cdiv(n + DUMMY_SPREAD, NS * 8) * NS * 8
    rpt = m // NS
    crows = pl.cdiv(pl.cdiv(m, D), 8) * 8
    mesh = plsc.VectorSubcoreMesh(
        core_axis_name="c", subcore_axis_name="s", num_cores=NC, num_subcores=NS
    )

    out_type = [jax.ShapeDtypeStruct((NC, m, D), jnp.float32)]
    scratch = [
        [pltpu.VMEM((C,), jnp.int32)] * 8,
        [pltpu.VMEM((C,), jnp.int32)] * 8,
        [pltpu.VMEM((C, D), jnp.float32)] * gring,
        [pltpu.VMEM((C, D), jnp.float32)] * gring,
        [pltpu.VMEM((C, D), jnp.float32)] * 2,
        pltpu.VMEM_SHARED((m, D), jnp.float32),
        [pltpu.SemaphoreType.DMA] * 8,
        [pltpu.SemaphoreType.DMA] * 8,
        [pltpu.SemaphoreType.DMA] * gring,
        [pltpu.SemaphoreType.DMA] * gring,
        [pltpu.SemaphoreType.DMA] * 2,
    ]
    if do_counts:
        out_type.append(jax.ShapeDtypeStruct((NC, crows, D), jnp.float32))
        scratch += [
            pltpu.VMEM((crows, D), jnp.float32),
            pltpu.VMEM((crows,), jnp.int32),
            pltpu.VMEM_SHARED((crows, D), jnp.float32),
        ]

    @functools.partial(
        pl.kernel,
        mesh=mesh,
        out_type=tuple(out_type),
        scratch_types=scratch,
        compiler_params=pltpu.CompilerParams(needs_layout_passes=False),
    )
    def edge_pass(a_hbm, b_hbm, src_hbm, dst_hbm, zz_hbm, out_hbm, *rest):
        if do_counts:
            (cnt_hbm, dsti, srci, ga, gb, gv, s_acc,
             sem_di, sem_si, sem_a, sem_b, sem_sc, cloc, rowids, c_acc) = rest
        else:
            (dsti, srci, ga, gb, gv, s_acc,
             sem_di, sem_si, sem_a, sem_b, sem_sc) = rest
        cid = lax.axis_index("c")
        sid = lax.axis_index("s")
        wid = sid * NC + cid

        zero16 = jnp.zeros((L,), jnp.float32)
        iota = lax.iota(jnp.int32, L)

        row0 = sid * rpt
        for k in range(rpt // ZROWS):
            pltpu.sync_copy(zz_hbm, s_acc.at[pl.ds(row0 + k * ZROWS, ZROWS)])
        zrem = rpt % ZROWS
        if zrem:
            pltpu.sync_copy(
                zz_hbm.at[pl.ds(0, zrem)],
                s_acc.at[pl.ds(row0 + (rpt // ZROWS) * ZROWS, zrem)],
            )
        if do_counts:
            def crow(i, carry):
                for j in range(D // L):
                    cloc[i, pl.ds(j * L, L)] = zero16
                return carry

            lax.fori_loop(0, crows, crow, 0)
            for k in range(crows // L):
                rowids[pl.ds(k * L, L)] = iota + k * L
            czslices = crows // 8
            @pl.when(sid < czslices)
            def _():
                pltpu.sync_copy(
                    zz_hbm.at[pl.ds(0, 8)], c_acc.at[pl.ds(8 * sid, 8)]
                )

        plsc.subcore_barrier()

        def fetch_idx(j, ib):
            @pl.when(j < chunks)
            def _():
                row = j * NW + wid
                pltpu.async_copy(dst_hbm.at[row], dsti[ib], sem_di[ib])
                pltpu.async_copy(src_hbm.at[row], srci[ib], sem_si[ib])

        def fetch_gather(j, ib, g):
            @pl.when(j < chunks)
            def _():
                pltpu.make_async_copy(dst_hbm.at[0], dsti[ib], sem_di[ib]).wait()
                pltpu.make_async_copy(src_hbm.at[0], srci[ib], sem_si[ib]).wait()
                pltpu.async_copy(a_hbm.at[dsti[ib]], ga[g], sem_a[g])
                pltpu.async_copy(b_hbm.at[srci[ib]], gb[g], sem_b[g])

        def wait_gather(ib, g):
            pltpu.make_async_copy(a_hbm.at[dsti[ib]], ga[g], sem_a[g]).wait()
            pltpu.make_async_copy(b_hbm.at[srci[ib]], gb[g], sem_b[g]).wait()

        def wait_scatter(ib, b):
            pltpu.make_async_copy(gv[b], s_acc.at[dsti[ib]], sem_sc[b]).wait()

        ones = jnp.ones((L,), jnp.float32)

        def process(j, ib, g, b):
            def row(i, inner):
                for j8 in range(D // L):
                    sl = pl.ds(j8 * L, L)
                    gv[b][i, sl] = jnp.maximum(ga[g][i, sl] + gb[g][i, sl], 0.0)
                return inner

            lax.fori_loop(0, C, row, 0)

            if do_counts:
                for k in range(C // L):
                    dv = dsti[ib][pl.ds(k * L, L)]
                    plsc.addupdate_scatter(
                        cloc,
                        [lax.shift_right_logical(dv, 7),
                         lax.bitwise_and(dv, 127)],
                        ones,
                    )
            pib, pb = (ib - 1) % 8, (b - 1) % 2
            @pl.when(j > 0)
            def _():
                wait_scatter(pib, pb)
            pltpu.async_copy(gv[b], s_acc.at[dsti[ib]], sem_sc[b], add=True)

        assert chunks % unroll == 1 and chunks > unroll
        assert gahead < gring and iahead - gahead >= 1 and iahead <= 6
        for p in range(iahead):
            fetch_idx(p, p)
        for p in range(gahead):
            fetch_gather(p, p, p)

        def pipe(kk, carry):
            j0 = kk * unroll
            for u in range(unroll):
                j = j0 + u
                ib, g, b = u % 8, u % gring, u % 2
                wait_gather(ib, g)
                fetch_idx(j + iahead, (u + iahead) % 8)
                fetch_gather(j + gahead, (u + gahead) % 8, (u + gahead) % gring)
                process(j, ib, g, b)
            return carry

        lax.fori_loop(0, chunks // unroll, pipe, 0)
        jt = (chunks // unroll) * unroll
        wait_gather(0, 0)
        process(jt, 0, 0, 0)
        wait_scatter(0, 0)

        if do_counts:
            pltpu.sync_copy(cloc, c_acc.at[rowids], add=True)
        plsc.subcore_barrier()

        pltpu.sync_copy(
            s_acc.at[pl.ds(row0, rpt)], out_hbm.at[cid, pl.ds(row0, rpt)]
        )
        if do_counts:
            @pl.when(sid < crows // 8)
            def _():
                pltpu.sync_copy(
                    c_acc.at[pl.ds(8 * sid, 8)],
                    cnt_hbm.at[cid, pl.ds(8 * sid, 8)],
                )

    return edge_pass, m, crows


def kernel(x, edge_index, W1a, b1a, W2a, b2a, W1b, b1b, W2b, b2b):
    n = x.shape[0]
    e = edge_index.shape[1]

    chunks = pl.cdiv(e, NW * C)
    while chunks % 24 != 1:
        chunks += 1
    e_pad = NW * C * chunks

    edge_a, m, crows = _make_edge_pass(
        n, chunks, do_counts=True, gring=3, gahead=2, iahead=4, unroll=24)
    edge_b, _, _ = _make_edge_pass(
        n, chunks, do_counts=False, gring=4, gahead=3, iahead=5, unroll=8)

    npad = e_pad - e
    src = jnp.concatenate(
        [edge_index[0], jnp.zeros((npad,), edge_index.dtype)]
    ).reshape(NW * chunks, C)
    dst = jnp.concatenate(
        [edge_index[1],
         (n + jnp.arange(npad, dtype=edge_index.dtype) % DUMMY_SPREAD)]
    ).reshape(NW * chunks, C)

    zz = jnp.zeros((ZROWS, D), jnp.float32)

    a1, bb1 = _encode(x, W1a, b1a, m)
    sp1, cnt1 = edge_a(a1, bb1, src, dst, zz)
    cnt_col = cnt1.reshape(NC, crows * D, 1)
    a2, bb2 = _mid(sp1, cnt_col, W2a, b2a, W1b, b1b, m)

    sp2 = edge_b(a2, bb2, src, dst, zz)
    if isinstance(sp2, (tuple, list)):
        (sp2,) = sp2
    return _decode(sp2, cnt_col, W2b, b2b, n)


# speedup vs baseline: 11.2764x; 1.0085x over previous
import functools

import jax
import jax.numpy as jnp
from jax import lax
from jax.experimental import pallas as pl
from jax.experimental.pallas import tpu as pltpu
from jax.experimental.pallas import tpu_sc as plsc

NC, NS, L = 2, 16, 16
NW = NC * NS

D = 128
C = 32
ZROWS = 64
DUMMY_SPREAD = 96


def _encode_body(x_ref, w_ref, b_ref, a_ref, bo_ref):
    xv = x_ref[...]
    a_ref[...] = (
        jnp.dot(xv, w_ref[0:D, :], preferred_element_type=jnp.float32)
        + b_ref[...]
    )
    bo_ref[...] = jnp.dot(xv, w_ref[D : 2 * D, :], preferred_element_type=jnp.float32)


def _encode(x, w1, b1, rows):
    bm = 416
    grid = (pl.cdiv(rows, bm),)
    return pl.pallas_call(
        _encode_body,
        grid=grid,
        in_specs=[
            pl.BlockSpec((bm, D), lambda i: (i, 0)),
            pl.BlockSpec((2 * D, D), lambda i: (0, 0)),
            pl.BlockSpec((1, D), lambda i: (0, 0)),
        ],
        out_specs=[
            pl.BlockSpec((bm, D), lambda i: (i, 0)),
            pl.BlockSpec((bm, D), lambda i: (i, 0)),
        ],
        out_shape=[
            jax.ShapeDtypeStruct((rows, D), jnp.float32),
            jax.ShapeDtypeStruct((rows, D), jnp.float32),
        ],
    )(x, w1, b1.reshape(1, D))


def _decode_body(sp_ref, cnt_ref, w_ref, b_ref, o_ref, *, apply_relu):
    s = sp_ref[0] + sp_ref[1]
    c = cnt_ref[0] + cnt_ref[1]
    t = jnp.dot(s, w_ref[...], preferred_element_type=jnp.float32)
    r = 1.0 / jnp.maximum(c, 1.0)
    o = t * r + jnp.where(c > 0.0, 1.0, 0.0) * b_ref[...]
    if apply_relu:
        o = jnp.maximum(o, 0.0)
    o_ref[...] = o


def _decode(sp, cnt_col, w2, b2, rows):
    bm = 416
    grid = (pl.cdiv(rows, bm),)
    return pl.pallas_call(
        functools.partial(_decode_body, apply_relu=False),
        grid=grid,
        in_specs=[
            pl.BlockSpec((2, bm, D), lambda i: (0, i, 0)),
            pl.BlockSpec((2, bm, 1), lambda i: (0, i, 0)),
            pl.BlockSpec((D, D), lambda i: (0, 0)),
            pl.BlockSpec((1, D), lambda i: (0, 0)),
        ],
        out_specs=pl.BlockSpec((bm, D), lambda i: (i, 0)),
        out_shape=jax.ShapeDtypeStruct((rows, D), jnp.float32),
    )(sp, cnt_col, w2, b2.reshape(1, D))


def _mid_body(sp_ref, cnt_ref, w2_ref, b2_ref, w1_ref, b1_ref, a_ref, bo_ref):
    s = sp_ref[0] + sp_ref[1]
    c = cnt_ref[0] + cnt_ref[1]
    t = jnp.dot(s, w2_ref[...], preferred_element_type=jnp.float32)
    r = 1.0 / jnp.maximum(c, 1.0)
    h = jnp.maximum(t * r + jnp.where(c > 0.0, 1.0, 0.0) * b2_ref[...], 0.0)
    a_ref[...] = (
        jnp.dot(h, w1_ref[0:D, :], preferred_element_type=jnp.float32)
        + b1_ref[...]
    )
    bo_ref[...] = jnp.dot(
        h, w1_ref[D : 2 * D, :], preferred_element_type=jnp.float32
    )


def _mid(sp, cnt_col, w2, b2, w1, b1, rows):
    bm = 416
    grid = (pl.cdiv(rows, bm),)
    return pl.pallas_call(
        _mid_body,
        grid=grid,
        in_specs=[
            pl.BlockSpec((2, bm, D), lambda i: (0, i, 0)),
            pl.BlockSpec((2, bm, 1), lambda i: (0, i, 0)),
            pl.BlockSpec((D, D), lambda i: (0, 0)),
            pl.BlockSpec((1, D), lambda i: (0, 0)),
            pl.BlockSpec((2 * D, D), lambda i: (0, 0)),
            pl.BlockSpec((1, D), lambda i: (0, 0)),
        ],
        out_specs=[
            pl.BlockSpec((bm, D), lambda i: (i, 0)),
            pl.BlockSpec((bm, D), lambda i: (i, 0)),
        ],
        out_shape=[
            jax.ShapeDtypeStruct((rows, D), jnp.float32),
            jax.ShapeDtypeStruct((rows, D), jnp.float32),
        ],
    )(sp, cnt_col, w2, b2.reshape(1, D), w1, b1.reshape(1, D))


def _make_edge_pass(n, chunks, do_counts, aring, ahead_a, bring, ahead_b,
                    iahead, unroll):
    m = pl.cdiv(n + DUMMY_SPREAD, NS * 8) * NS * 8
    rpt = m // NS
    crows = pl.cdiv(pl.cdiv(m, D), 8) * 8
    mesh = plsc.VectorSubcoreMesh(
        core_axis_name="c", subcore_axis_name="s", num_cores=NC, num_subcores=NS
    )

    out_type = [jax.ShapeDtypeStruct((NC, m, D), jnp.float32)]
    scratch = [
        [pltpu.VMEM((C,), jnp.int32)] * 8,
        [pltpu.VMEM((C,), jnp.int32)] * 8,
        [pltpu.VMEM((C, D), jnp.float32)] * aring,
        [pltpu.VMEM((C, D), jnp.float32)] * bring,
        [pltpu.VMEM((C, D), jnp.float32)] * 2,
        pltpu.VMEM_SHARED((m, D), jnp.float32),
        [pltpu.SemaphoreType.DMA] * 8,
        [pltpu.SemaphoreType.DMA] * 8,
        [pltpu.SemaphoreType.DMA] * aring,
        [pltpu.SemaphoreType.DMA] * bring,
        [pltpu.SemaphoreType.DMA] * 2,
    ]
    if do_counts:
        out_type.append(jax.ShapeDtypeStruct((NC, crows, D), jnp.float32))
        scratch += [
            pltpu.VMEM((crows, D), jnp.float32),
            pltpu.VMEM((crows,), jnp.int32),
            pltpu.VMEM_SHARED((crows, D), jnp.float32),
        ]

    @functools.partial(
        pl.kernel,
        mesh=mesh,
        out_type=tuple(out_type),
        scratch_types=scratch,
        compiler_params=pltpu.CompilerParams(needs_layout_passes=False),
    )
    def edge_pass(a_hbm, b_hbm, src_hbm, dst_hbm, zz_hbm, out_hbm, *rest):
        if do_counts:
            (cnt_hbm, dsti, srci, ga, gb, gv, s_acc,
             sem_di, sem_si, sem_a, sem_b, sem_sc, cloc, rowids, c_acc) = rest
        else:
            (dsti, srci, ga, gb, gv, s_acc,
             sem_di, sem_si, sem_a, sem_b, sem_sc) = rest
        cid = lax.axis_index("c")
        sid = lax.axis_index("s")
        wid = sid * NC + cid

        zero16 = jnp.zeros((L,), jnp.float32)
        iota = lax.iota(jnp.int32, L)

        row0 = sid * rpt
        for k in range(rpt // ZROWS):
            pltpu.sync_copy(zz_hbm, s_acc.at[pl.ds(row0 + k * ZROWS, ZROWS)])
        zrem = rpt % ZROWS
        if zrem:
            pltpu.sync_copy(
                zz_hbm.at[pl.ds(0, zrem)],
                s_acc.at[pl.ds(row0 + (rpt // ZROWS) * ZROWS, zrem)],
            )
        if do_counts:
            def crow(i, carry):
                for j in range(D // L):
                    cloc[i, pl.ds(j * L, L)] = zero16
                return carry

            lax.fori_loop(0, crows, crow, 0)
            for k in range(crows // L):
                rowids[pl.ds(k * L, L)] = iota + k * L
            czslices = crows // 8
            @pl.when(sid < czslices)
            def _():
                pltpu.sync_copy(
                    zz_hbm.at[pl.ds(0, 8)], c_acc.at[pl.ds(8 * sid, 8)]
                )

        plsc.subcore_barrier()

        def fetch_idx(j, ib):
            @pl.when(j < chunks)
            def _():
                row = j * NW + wid
                pltpu.async_copy(dst_hbm.at[row], dsti[ib], sem_di[ib])
                pltpu.async_copy(src_hbm.at[row], srci[ib], sem_si[ib])

        def fetch_ga(j, ib, g):
            @pl.when(j < chunks)
            def _():
                pltpu.make_async_copy(dst_hbm.at[0], dsti[ib], sem_di[ib]).wait()
                pltpu.async_copy(a_hbm.at[dsti[ib]], ga[g], sem_a[g])

        def fetch_gb(j, ib, g):
            @pl.when(j < chunks)
            def _():
                pltpu.make_async_copy(src_hbm.at[0], srci[ib], sem_si[ib]).wait()
                pltpu.async_copy(b_hbm.at[srci[ib]], gb[g], sem_b[g])

        def wait_gather(ib, ag, bg):
            pltpu.make_async_copy(a_hbm.at[dsti[ib]], ga[ag], sem_a[ag]).wait()
            pltpu.make_async_copy(b_hbm.at[srci[ib]], gb[bg], sem_b[bg]).wait()

        def wait_scatter(ib, b):
            pltpu.make_async_copy(gv[b], s_acc.at[dsti[ib]], sem_sc[b]).wait()

        ones = jnp.ones((L,), jnp.float32)

        def process(j, ib, ag, bg, b):
            def row(i, inner):
                for j8 in range(D // L):
                    sl = pl.ds(j8 * L, L)
                    gv[b][i, sl] = jnp.maximum(
                        ga[ag][i, sl] + gb[bg][i, sl], 0.0)
                return inner

            lax.fori_loop(0, C, row, 0)

            if do_counts:
                for k in range(C // L):
                    dv = dsti[ib][pl.ds(k * L, L)]
                    plsc.addupdate_scatter(
                        cloc,
                        [lax.shift_right_logical(dv, 7),
                         lax.bitwise_and(dv, 127)],
                        ones,
                    )
            pib, pb = (ib - 1) % 8, (b - 1) % 2
            @pl.when(j > 0)
            def _():
                wait_scatter(pib, pb)
            pltpu.async_copy(gv[b], s_acc.at[dsti[ib]], sem_sc[b], add=True)

        assert chunks % unroll == 1 and chunks > unroll
        assert ahead_a < aring and ahead_b < bring
        assert iahead > max(ahead_a, ahead_b) and iahead <= 6
        for p in range(iahead):
            fetch_idx(p, p)
        for p in range(ahead_a):
            fetch_ga(p, p, p)
        for p in range(ahead_b):
            fetch_gb(p, p, p)

        def pipe(kk, carry):
            j0 = kk * unroll
            for u in range(unroll):
                j = j0 + u
                ib, ag, bg, b = u % 8, u % aring, u % bring, u % 2
                wait_gather(ib, ag, bg)
                fetch_idx(j + iahead, (u + iahead) % 8)
                fetch_ga(j + ahead_a, (u + ahead_a) % 8, (u + ahead_a) % aring)
                fetch_gb(j + ahead_b, (u + ahead_b) % 8, (u + ahead_b) % bring)
                process(j, ib, ag, bg, b)
            return carry

        lax.fori_loop(0, chunks // unroll, pipe, 0)
        jt = (chunks // unroll) * unroll
        wait_gather(0, 0, 0)
        process(jt, 0, 0, 0, 0)
        wait_scatter(0, 0)

        if do_counts:
            pltpu.sync_copy(cloc, c_acc.at[rowids], add=True)
        plsc.subcore_barrier()

        pltpu.sync_copy(
            s_acc.at[pl.ds(row0, rpt)], out_hbm.at[cid, pl.ds(row0, rpt)]
        )
        if do_counts:
            @pl.when(sid < crows // 8)
            def _():
                pltpu.sync_copy(
                    c_acc.at[pl.ds(8 * sid, 8)],
                    cnt_hbm.at[cid, pl.ds(8 * sid, 8)],
                )

    return edge_pass, m, crows


def kernel(x, edge_index, W1a, b1a, W2a, b2a, W1b, b1b, W2b, b2b):
    n = x.shape[0]
    e = edge_index.shape[1]

    chunks = pl.cdiv(e, NW * C)
    while chunks % 24 != 1:
        chunks += 1
    e_pad = NW * C * chunks

    edge_a, m, crows = _make_edge_pass(
        n, chunks, do_counts=True, aring=4, ahead_a=3, bring=3, ahead_b=2,
        iahead=5, unroll=24)
    edge_b, _, _ = _make_edge_pass(
        n, chunks, do_counts=False, aring=4, ahead_a=3, bring=4, ahead_b=3,
        iahead=5, unroll=8)

    npad = e_pad - e
    src = jnp.concatenate(
        [edge_index[0], jnp.zeros((npad,), edge_index.dtype)]
    ).reshape(NW * chunks, C)
    dst = jnp.concatenate(
        [edge_index[1],
         (n + jnp.arange(npad, dtype=edge_index.dtype) % DUMMY_SPREAD)]
    ).reshape(NW * chunks, C)

    zz = jnp.zeros((ZROWS, D), jnp.float32)

    a1, bb1 = _encode(x, W1a, b1a, m)
    sp1, cnt1 = edge_a(a1, bb1, src, dst, zz)
    cnt_col = cnt1.reshape(NC, crows * D, 1)
    a2, bb2 = _mid(sp1, cnt_col, W2a, b2a, W1b, b1b, m)

    sp2 = edge_b(a2, bb2, src, dst, zz)
    if isinstance(sp2, (tuple, list)):
        (sp2,) = sp2
    return _decode(sp2, cnt_col, W2b, b2b, n)
